# Initial kernel scaffold; baseline (speedup 1.0000x reference)
import jax
import jax.numpy as jnp
from jax.experimental import pallas as pl


def kernel(features, edge_weights, threashold, Wl1, bl1, Wr1, br1, We1, att1, bo1, Wl2, bl2, Wr2, br2, We2, att2, bo2, Wl3, bl3, Wr3, br3, We3, att3, bo3):
    raise NotImplementedError("write your pallas kernel here")


# speedup vs baseline: 136.9042x; 136.9042x over previous
import functools

import jax
import jax.numpy as jnp
from jax.experimental import pallas as pl
from jax.experimental.pallas import tpu as pltpu

_N = 1024
_D = 128
_TBLK = 128
_SCHUNK = 128
_TSUB = 8

_HIGH = jax.lax.Precision.HIGHEST


def _lin_body(x_ref, wlt_ref, bl_ref, wrt_ref, br_ref, xl_ref, xr_ref):
    x = x_ref[...]
    xl_ref[...] = jnp.dot(x, wlt_ref[...],
                          preferred_element_type=jnp.float32) + bl_ref[...]
    xr_ref[...] = jnp.dot(x, wrt_ref[...],
                          preferred_element_type=jnp.float32) + br_ref[...]


def _linear(x, WlT, bl, WrT, br):
    xl, xr = pl.pallas_call(
        _lin_body,
        out_shape=[jax.ShapeDtypeStruct((_N, _D), jnp.float32)] * 2,
    )(x, WlT, bl.reshape(1, _D), WrT, br.reshape(1, _D))
    return xl, xr


def _attn_body(normalize, xl_ref, xr_ref, ew_ref, we_ref, att_ref, bo_ref,
               cut_ref, o_ref, alpha_ref):
    cut = cut_ref[0, 0]
    xr = xr_ref[...]
    we = we_ref[...].reshape(1, 1, _D)
    att = att_ref[...].reshape(1, 1, _D)

    def chunk(si, carry):
        s0 = si * _SCHUNK
        xl_c = xl_ref[pl.ds(s0, _SCHUNK), :]
        for tc in range(_TBLK // _TSUB):
            t0 = tc * _TSUB
            ew_t = ew_ref[pl.ds(s0, _SCHUNK), tc, :]
            z = (xl_c[:, None, :] + xr[t0:t0 + _TSUB][None, :, :]
                 + ew_t[:, :, None] * we)
            m = jnp.maximum(z, 0.2 * z)
            p = m * att
            alpha_ref[pl.ds(s0, _SCHUNK), tc, :] = jnp.sum(p, axis=-1)
        return carry

    jax.lax.fori_loop(0, _N // _SCHUNK, chunk, 0)

    ew = ew_ref[...]
    alpha = jnp.where(ew > cut, alpha_ref[...], -jnp.inf)
    amax = jnp.max(alpha, axis=0, keepdims=True)
    amax = jnp.where(jnp.isfinite(amax), amax, 0.0)
    ex = jnp.exp(alpha - amax)
    den = jnp.sum(ex, axis=0, keepdims=True)
    a = (ex / (den + 1e-16)).reshape(_N, _TBLK)
    out = jax.lax.dot_general(a, xl_ref[...], (((0,), (0,)), ((), ())),
                              preferred_element_type=jnp.float32,
                              precision=_HIGH)
    out = out + bo_ref[...]
    if normalize:
        out = out / jnp.sum(out, axis=1, keepdims=True)
    o_ref[...] = out


def _attn(xl, xr, ew4, We, att, bo, cut, normalize):
    return pl.pallas_call(
        functools.partial(_attn_body, normalize),
        grid=(_N // _TBLK,),
        in_specs=[
            pl.BlockSpec((_N, _D), lambda i: (0, 0)),
            pl.BlockSpec((_TBLK, _D), lambda i: (i, 0)),
            pl.BlockSpec((_N, _TBLK // _TSUB, _TSUB),
                         lambda i: (0, i, 0)),
            pl.BlockSpec((1, _D), lambda i: (0, 0)),
            pl.BlockSpec((1, _D), lambda i: (0, 0)),
            pl.BlockSpec((1, _D), lambda i: (0, 0)),
            pl.BlockSpec((1, 1), lambda i: (0, 0)),
        ],
        out_specs=pl.BlockSpec((_TBLK, _D), lambda i: (i, 0)),
        out_shape=jax.ShapeDtypeStruct((_N, _D), jnp.float32),
        scratch_shapes=[pltpu.VMEM((_N, _TBLK // _TSUB, _TSUB), jnp.float32)],
    )(xl, xr, ew4, We.reshape(1, _D), att.reshape(1, _D), bo.reshape(1, _D),
      cut)


def _l3_body(x_ref, ew_ref, wlt_ref, bl_ref, wrt_ref, br_ref, we_ref,
             att_ref, bo_ref, cut_ref, o_ref):
    x = x_ref[...]
    xl = jnp.dot(x, wlt_ref[...],
                 preferred_element_type=jnp.float32) + bl_ref[0, 0]
    xr = jnp.dot(x, wrt_ref[...],
                 preferred_element_type=jnp.float32) + br_ref[0, 0]
    xr_row = jax.lax.dot_general(jnp.ones((1, 1), jnp.float32), xr,
                                 (((1,), (1,)), ((), ())),
                                 preferred_element_type=jnp.float32,
                                 precision=_HIGH)
    ew = ew_ref[...]
    z = xl + xr_row + ew * we_ref[0, 0]
    m = jnp.maximum(z, 0.2 * z)
    alpha = m * att_ref[0, 0]
    alpha = jnp.where(ew > cut_ref[0, 0], alpha, -jnp.inf)
    amax = jnp.max(alpha, axis=0, keepdims=True)
    amax = jnp.where(jnp.isfinite(amax), amax, 0.0)
    ex = jnp.exp(alpha - amax)
    den = jnp.sum(ex, axis=0, keepdims=True)
    a = ex / (den + 1e-16)
    x3 = jnp.sum(a * xl, axis=0, keepdims=True) + bo_ref[0, 0]
    o_ref[...] = jnp.sum(x3, axis=1, keepdims=True) * (1.0 / _N)


def _layer3(x, ew, WlT, bl, WrT, br, We, att, bo, cut):
    return pl.pallas_call(
        _l3_body,
        out_shape=jax.ShapeDtypeStruct((1, 1), jnp.float32),
    )(x, ew, WlT, bl.reshape(1, 1), WrT, br.reshape(1, 1),
      We.reshape(1, 1), att.reshape(1, 1), bo.reshape(1, 1), cut)


def kernel(features, edge_weights, threashold, Wl1, bl1, Wr1, br1, We1, att1,
           bo1, Wl2, bl2, Wr2, br2, We2, att2, bo2, Wl3, bl3, Wr3, br3, We3,
           att3, bo3):
    cut = (1.0 / threashold) * jnp.ones((1, 1), jnp.float32)
    ew4 = edge_weights.reshape(_N, _N // _TSUB, _TSUB)

    xl1, xr1 = _linear(features, Wl1.T, bl1, Wr1.T, br1)
    x1 = _attn(xl1, xr1, ew4, We1, att1, bo1, cut, normalize=True)

    xl2, xr2 = _linear(x1, Wl2.T, bl2, Wr2.T, br2)
    x2 = _attn(xl2, xr2, ew4, We2, att2, bo2, cut, normalize=True)

    x4 = _layer3(x2, edge_weights, Wl3.T, bl3, Wr3.T, br3, We3, att3, bo3,
                 cut)
    return x4.reshape(1)
